# baseline (speedup 1.0000x reference)
import functools

import jax
import jax.numpy as jnp
from jax import lax
from jax.experimental import pallas as pl
from jax.experimental.pallas import tpu as pltpu
from jax.experimental.pallas import tpu_sc as plsc

N_GRID = 64800
N_MESH = 10242
M_PAD = 10368
W, H = 180, 360
HID = 128
F32 = jnp.float32

NC, NS = 2, 16
NW = NC * NS


def _ln(h, g, b):
    mu = jnp.mean(h, axis=-1, keepdims=True)
    var = jnp.mean((h - mu) ** 2, axis=-1, keepdims=True)
    return (h - mu) * lax.rsqrt(var + 1e-5) * g + b


def _full(shape):
    return pl.BlockSpec(shape, lambda i: (0,) * len(shape))


def _rows(bsize, ncol):
    return pl.BlockSpec((bsize, ncol), lambda i: (i, 0))


def _vec(v):
    return v.reshape(1, -1)


def _embed(x, p, bsize):
    n, din = x.shape
    dp = -(-din // 8) * 8
    if dp != din:
        x = jnp.pad(x, ((0, 0), (0, dp - din)))
        w1 = jnp.pad(p["w1"], ((0, dp - din), (0, 0)))
    else:
        w1 = p["w1"]

    def kern(x_ref, w1_ref, b1_ref, w2_ref, b2_ref, g_ref, bb_ref, o_ref):
        h = jax.nn.silu(
            jnp.dot(x_ref[...], w1_ref[...], preferred_element_type=F32)
            + b1_ref[...])
        h = jnp.dot(h, w2_ref[...], preferred_element_type=F32) + b2_ref[...]
        o_ref[...] = _ln(h, g_ref[...], bb_ref[...])

    return pl.pallas_call(
        kern,
        grid=(n // bsize,),
        in_specs=[_rows(bsize, dp), _full((dp, HID)), _full((1, HID)),
                  _full((HID, HID)), _full((1, HID)), _full((1, HID)),
                  _full((1, HID))],
        out_specs=_rows(bsize, HID),
        out_shape=jax.ShapeDtypeStruct((n, HID), F32),
    )(x, w1, _vec(p["b1"]), p["w2"], _vec(p["b2"]), _vec(p["ln_g"]),
      _vec(p["ln_b"]))


def _res_mlp(x, p, bsize):
    n = x.shape[0]

    def kern(x_ref, w1_ref, b1_ref, w2_ref, b2_ref, g_ref, bb_ref, o_ref):
        xv = x_ref[...]
        h = jax.nn.silu(
            jnp.dot(xv, w1_ref[...], preferred_element_type=F32) + b1_ref[...])
        h = jnp.dot(h, w2_ref[...], preferred_element_type=F32) + b2_ref[...]
        o_ref[...] = _ln(h, g_ref[...], bb_ref[...]) + xv

    return pl.pallas_call(
        kern,
        grid=(n // bsize,),
        in_specs=[_rows(bsize, HID), _full((HID, HID)), _full((1, HID)),
                  _full((HID, HID)), _full((1, HID)), _full((1, HID)),
                  _full((1, HID))],
        out_specs=_rows(bsize, HID),
        out_shape=jax.ShapeDtypeStruct((n, HID), F32),
    )(x, p["w1"], _vec(p["b1"]), p["w2"], _vec(p["b2"]), _vec(p["ln_g"]),
      _vec(p["ln_b"]))


def _proj(x, w, bsize):
    n = x.shape[0]

    def kern(x_ref, w_ref, o_ref):
        o_ref[...] = jnp.dot(x_ref[...], w_ref[...], preferred_element_type=F32)

    return pl.pallas_call(
        kern,
        grid=(n // bsize,),
        in_specs=[_rows(bsize, HID), _full((HID, HID))],
        out_specs=_rows(bsize, HID),
        out_shape=jax.ShapeDtypeStruct((n, HID), F32),
    )(x, w)


def _proj2(x, wa, wb, bsize):
    n = x.shape[0]

    def kern(x_ref, wa_ref, wb_ref, oa_ref, ob_ref):
        xv = x_ref[...]
        oa_ref[...] = jnp.dot(xv, wa_ref[...], preferred_element_type=F32)
        ob_ref[...] = jnp.dot(xv, wb_ref[...], preferred_element_type=F32)

    return pl.pallas_call(
        kern,
        grid=(n // bsize,),
        in_specs=[_rows(bsize, HID), _full((HID, HID)), _full((HID, HID))],
        out_specs=[_rows(bsize, HID), _rows(bsize, HID)],
        out_shape=[jax.ShapeDtypeStruct((n, HID), F32),
                   jax.ShapeDtypeStruct((n, HID), F32)],
    )(x, wa, wb)


def _edge_stage(e, gsum, p, bsize):
    n = e.shape[0]
    w1a = p["w1"][:HID]

    def kern(e_ref, gs_ref, w1_ref, b1_ref, w2_ref, b2_ref, g_ref, bb_ref,
             o_ref):
        ev = e_ref[...]
        pre = (jnp.dot(ev, w1_ref[...], preferred_element_type=F32)
               + gs_ref[...] + b1_ref[...])
        h = (jnp.dot(jax.nn.silu(pre), w2_ref[...], preferred_element_type=F32)
             + b2_ref[...])
        o_ref[...] = _ln(h, g_ref[...], bb_ref[...]) + ev

    return pl.pallas_call(
        kern,
        grid=(n // bsize,),
        in_specs=[_rows(bsize, HID), _rows(bsize, HID), _full((HID, HID)),
                  _full((1, HID)), _full((HID, HID)), _full((1, HID)),
                  _full((1, HID)), _full((1, HID))],
        out_specs=_rows(bsize, HID),
        out_shape=jax.ShapeDtypeStruct((n, HID), F32),
    )(e, gsum, w1a, _vec(p["b1"]), p["w2"], _vec(p["b2"]), _vec(p["ln_g"]),
      _vec(p["ln_b"]))


def _node_stage(aggs, node, p, bsize):
    n = node.shape[0]
    nagg = len(aggs)
    w1a = p["w1"][:HID]
    w1b = p["w1"][HID:]

    def kern(*refs):
        agg_refs = refs[:nagg]
        (node_ref, w1a_ref, w1b_ref, b1_ref, w2_ref, b2_ref, g_ref, bb_ref,
         o_ref) = refs[nagg:]
        a = agg_refs[0][...]
        for r in agg_refs[1:]:
            a = a + r[...]
        nv = node_ref[...]
        pre = (jnp.dot(a, w1a_ref[...], preferred_element_type=F32)
               + jnp.dot(nv, w1b_ref[...], preferred_element_type=F32)
               + b1_ref[...])
        h = (jnp.dot(jax.nn.silu(pre), w2_ref[...], preferred_element_type=F32)
             + b2_ref[...])
        o_ref[...] = _ln(h, g_ref[...], bb_ref[...]) + nv

    return pl.pallas_call(
        kern,
        grid=(n // bsize,),
        in_specs=[_rows(bsize, HID)] * nagg
        + [_rows(bsize, HID), _full((HID, HID)), _full((HID, HID)),
           _full((1, HID)), _full((HID, HID)), _full((1, HID)),
           _full((1, HID)), _full((1, HID))],
        out_specs=_rows(bsize, HID),
        out_shape=jax.ShapeDtypeStruct((n, HID), F32),
    )(*aggs, node, w1a, w1b, _vec(p["b1"]), p["w2"], _vec(p["b2"]),
      _vec(p["ln_g"]), _vec(p["ln_b"]))


def _edge_stage_split(e, gsum, p, bsize, ngrp):
    n = e.shape[0]
    w1a = p["w1"][:HID]
    cw = HID // ngrp

    def kern(*refs):
        e_ref, gs_ref, w1_ref, b1_ref, w2_ref, b2_ref, g_ref, bb_ref = refs[:8]
        outs = refs[8:]
        ev = e_ref[...]
        pre = (jnp.dot(ev, w1_ref[...], preferred_element_type=F32)
               + gs_ref[...] + b1_ref[...])
        h = (jnp.dot(jax.nn.silu(pre), w2_ref[...], preferred_element_type=F32)
             + b2_ref[...])
        res = _ln(h, g_ref[...], bb_ref[...]) + ev
        for gi, o in enumerate(outs):
            o[...] = res[:, gi * cw:(gi + 1) * cw]

    return pl.pallas_call(
        kern,
        grid=(n // bsize,),
        in_specs=[_rows(bsize, HID), _rows(bsize, HID), _full((HID, HID)),
                  _full((1, HID)), _full((HID, HID)), _full((1, HID)),
                  _full((1, HID)), _full((1, HID))],
        out_specs=[_rows(bsize, cw)] * ngrp,
        out_shape=[jax.ShapeDtypeStruct((n, cw), F32)] * ngrp,
    )(e, gsum, w1a, _vec(p["b1"]), p["w2"], _vec(p["b2"]), _vec(p["ln_g"]),
      _vec(p["ln_b"]))


def _node_stage_cat(parts, node, p, bsize):
    n = node.shape[0]
    npart = len(parts)
    cw = HID // npart
    w1a = p["w1"][:HID]
    w1b = p["w1"][HID:]

    def kern(*refs):
        agg_refs = refs[:npart]
        (node_ref, w1a_ref, w1b_ref, b1_ref, w2_ref, b2_ref, g_ref, bb_ref,
         o_ref) = refs[npart:]
        a = jnp.concatenate([r[...] for r in agg_refs], axis=-1)
        nv = node_ref[...]
        pre = (jnp.dot(a, w1a_ref[...], preferred_element_type=F32)
               + jnp.dot(nv, w1b_ref[...], preferred_element_type=F32)
               + b1_ref[...])
        h = (jnp.dot(jax.nn.silu(pre), w2_ref[...], preferred_element_type=F32)
             + b2_ref[...])
        o_ref[...] = _ln(h, g_ref[...], bb_ref[...]) + nv

    return pl.pallas_call(
        kern,
        grid=(n // bsize,),
        in_specs=[_rows(bsize, cw)] * npart
        + [_rows(bsize, HID), _full((HID, HID)), _full((HID, HID)),
           _full((1, HID)), _full((HID, HID)), _full((1, HID)),
           _full((1, HID)), _full((1, HID))],
        out_specs=_rows(bsize, HID),
        out_shape=jax.ShapeDtypeStruct((n, HID), F32),
    )(*parts, node, w1a, w1b, _vec(p["b1"]), p["w2"], _vec(p["b2"]),
      _vec(p["ln_g"]), _vec(p["ln_b"]))


def _decoder(x, p, bsize):
    n = x.shape[0]
    w2 = jnp.pad(p["w2"], ((0, 0), (0, 7)))
    b2 = jnp.pad(p["b2"], ((0, 7),))

    def kern(x_ref, w1_ref, b1_ref, w2_ref, b2_ref, o_ref):
        h = jax.nn.silu(
            jnp.dot(x_ref[...], w1_ref[...], preferred_element_type=F32)
            + b1_ref[...])
        o_ref[...] = (jnp.dot(h, w2_ref[...], preferred_element_type=F32)
                      + b2_ref[...])

    return pl.pallas_call(
        kern,
        grid=(n // bsize,),
        in_specs=[_rows(bsize, HID), _full((HID, HID)), _full((1, HID)),
                  _full((HID, 8)), _full((1, 8))],
        out_specs=_rows(bsize, 8),
        out_shape=jax.ShapeDtypeStruct((n, 8), F32),
    )(x, p["w1"], _vec(p["b1"]), w2, _vec(b2))


def _sc_gather_sum(tab_a, tab_b, idx_a, idx_b, chunk):
    ne = idx_a.shape[0]
    nchunks = ne // chunk
    kmax = -(-nchunks // NW)
    iters = -(-kmax // 2)
    mesh = plsc.VectorSubcoreMesh(core_axis_name="c", subcore_axis_name="s")

    @functools.partial(
        pl.kernel,
        out_type=jax.ShapeDtypeStruct((ne, HID), F32),
        mesh=mesh,
        scratch_types=[
            pltpu.VMEM((chunk,), jnp.int32),
            pltpu.VMEM((chunk,), jnp.int32),
            pltpu.VMEM((chunk, HID), F32),
            pltpu.VMEM((chunk, HID), F32),
            pltpu.VMEM((chunk,), jnp.int32),
            pltpu.VMEM((chunk,), jnp.int32),
            pltpu.VMEM((chunk, HID), F32),
            pltpu.VMEM((chunk, HID), F32),
            pltpu.SemaphoreType.DMA,
            pltpu.SemaphoreType.DMA,
            pltpu.SemaphoreType.DMA,
            pltpu.SemaphoreType.DMA,
        ],
    )
    def k(ta, tb, ia, ib, out,
          ia0, ib0, ba0, bb0, ia1, ib1, ba1, bb1, sa0, sb0, sa1, sb1):
        wid = lax.axis_index("s") * NC + lax.axis_index("c")
        iav, ibv = (ia0, ia1), (ib0, ib1)
        bav, bbv = (ba0, ba1), (bb0, bb1)
        sav, sbv = (sa0, sa1), (sb0, sb1)

        def start(slot, kk):
            cid = wid + kk * NW

            @pl.when(cid < nchunks)
            def _():
                base = cid * chunk
                pltpu.sync_copy(ia.at[pl.ds(base, chunk)], iav[slot])
                pltpu.sync_copy(ib.at[pl.ds(base, chunk)], ibv[slot])
                pltpu.async_copy(ta.at[iav[slot]], bav[slot], sav[slot])
                pltpu.async_copy(tb.at[ibv[slot]], bbv[slot], sbv[slot])

        def proc(slot, kk):
            cid = wid + kk * NW

            @pl.when(cid < nchunks)
            def _():
                base = cid * chunk
                pltpu.make_async_copy(
                    ta.at[iav[slot]], bav[slot], sav[slot]).wait()
                pltpu.make_async_copy(
                    tb.at[ibv[slot]], bbv[slot], sbv[slot]).wait()
                ba, bb = bav[slot], bbv[slot]

                @plsc.parallel_loop(0, chunk, unroll=4)
                def _add(i):
                    for j in range(HID // 16):
                        sl = pl.ds(j * 16, 16)
                        ba[i, sl] = ba[i, sl] + bb[i, sl]

                pltpu.sync_copy(ba, out.at[pl.ds(base, chunk)])

        start(0, 0)

        def step(it, carry):
            k0 = 2 * it
            start(1, k0 + 1)
            proc(0, k0)
            start(0, k0 + 2)
            proc(1, k0 + 1)
            return carry

        lax.fori_loop(0, iters, step, 0)

    return k(tab_a, tab_b, idx_a, idx_b)


def _sc_segsum_mesh(vals, dst, chunk):
    ne = vals.shape[0]
    nchunks = ne // chunk
    iters = -(-(-(-nchunks // 2)) // NS)
    nzc = M_PAD // 128
    ziters = -(-nzc // NS)
    mesh = plsc.VectorSubcoreMesh(core_axis_name="c", subcore_axis_name="s")

    @functools.partial(
        pl.kernel,
        out_type=jax.ShapeDtypeStruct((2 * M_PAD, HID), F32),
        mesh=mesh,
        scratch_types=[
            pltpu.VMEM((chunk,), jnp.int32),
            pltpu.VMEM((chunk, HID), F32),
            pltpu.VMEM((128, HID), F32),
            pltpu.VMEM_SHARED((M_PAD, HID), F32),
        ],
    )
    def k(vals_h, dst_h, out_h, idx_v, buf_v, zb_v, acc_s):
        c = lax.axis_index("c")
        s = lax.axis_index("s")

        def zrow(i, c2):
            for j in range(HID // 16):
                zb_v[i, pl.ds(j * 16, 16)] = jnp.zeros((16,), F32)
            return c2

        lax.fori_loop(0, 128, zrow, 0)

        def zstep(it, c2):
            zc = s + it * NS

            @pl.when(zc < nzc)
            def _():
                pltpu.sync_copy(zb_v, acc_s.at[pl.ds(zc * 128, 128)])

            return c2

        lax.fori_loop(0, ziters, zstep, 0)
        plsc.subcore_barrier()

        def step(it, c2):
            cid = c + 2 * (s + it * NS)

            @pl.when(cid < nchunks)
            def _():
                base = cid * chunk
                pltpu.sync_copy(dst_h.at[pl.ds(base, chunk)], idx_v)
                pltpu.sync_copy(vals_h.at[pl.ds(base, chunk)], buf_v)
                pltpu.sync_copy(buf_v, acc_s.at[idx_v], add=True)

            return c2

        lax.fori_loop(0, iters, step, 0)
        plsc.subcore_barrier()

        def wstep(it, c2):
            zc = s + it * NS

            @pl.when(zc < nzc)
            def _():
                pltpu.sync_copy(acc_s.at[pl.ds(zc * 128, 128)], zb_v)
                pltpu.sync_copy(
                    zb_v, out_h.at[pl.ds(c * M_PAD + zc * 128, 128)])

            return c2

        lax.fori_loop(0, ziters, wstep, 0)

    return k(vals, dst)


_GR_RANGE = 32400
_GR_SP = 32512
_GR_DUMMY = 32400
_GR_WB = 144
_NGRP = 8
_CW = HID // _NGRP


def _sc_segsum_grid(vals8, dst, chunk, grp):
    ne = dst.shape[0]
    sup = chunk * grp
    nsuper = ne // sup
    assert nsuper * sup == ne
    iters = -(-nsuper // NS)
    nzc = _GR_SP // 128
    ziters = -(-nzc // NS)
    nwb = _GR_RANGE // _GR_WB
    witers = -(-nwb // NS)
    mesh = plsc.VectorSubcoreMesh(core_axis_name="c", subcore_axis_name="s")

    @functools.partial(
        pl.kernel,
        out_type=jax.ShapeDtypeStruct((_NGRP * N_GRID, _CW), F32),
        mesh=mesh,
        compiler_params=pltpu.CompilerParams(use_tc_tiling_on_sc=False),
        scratch_types=[
            pltpu.VMEM((sup,), jnp.int32),
            pltpu.VMEM((iters * grp, chunk), jnp.int32),
            pltpu.VMEM((sup, _CW), F32),
            pltpu.VMEM((128, _CW), F32),
            pltpu.VMEM((_GR_WB, _CW), F32),
            pltpu.VMEM_SHARED((_GR_SP, _CW), F32),
            pltpu.SemaphoreType.DMA,
        ],
    )
    def k(v0, v1, v2, v3, v4, v5, v6, v7, dst_h, out_h,
          idx_v, rel_v, buf_v, zb_v, wb_v, acc_s, sem):
        vs = (v0, v1, v2, v3, v4, v5, v6, v7)
        c = lax.axis_index("c")
        s = lax.axis_index("s")
        rbase = c * _GR_RANGE

        def zrow(i, c2):
            zb_v[i, pl.ds(0, 16)] = jnp.zeros((16,), F32)
            return c2

        lax.fori_loop(0, 128, zrow, 0)

        def pre(it, c2):
            sid = s + it * NS

            @pl.when(sid < nsuper)
            def _():
                base = sid * sup
                pltpu.sync_copy(dst_h.at[pl.ds(base, sup)], idx_v)

                def remap(r, c3):
                    for j in range(grp):
                        v = idx_v[pl.ds(j * chunk + r * 16, 16)] - rbase
                        ok = (v >= 0) & (v < _GR_RANGE)
                        rel_v[it * grp + j, pl.ds(r * 16, 16)] = jnp.where(
                            ok, v, _GR_DUMMY)
                    return c3

                lax.fori_loop(0, chunk // 16, remap, 0)

            return c2

        lax.fori_loop(0, iters, pre, 0)

        for g in range(_NGRP):
            def zstep(it, c2):
                zc = s + it * NS

                @pl.when(zc < nzc)
                def _():
                    pltpu.sync_copy(zb_v, acc_s.at[pl.ds(zc * 128, 128)])

                return c2

            lax.fori_loop(0, ziters, zstep, 0)
            plsc.subcore_barrier()

            def step(it, c2):
                sid = s + it * NS

                @pl.when(sid < nsuper)
                def _():
                    base = sid * sup
                    pltpu.async_copy(
                        vs[g].at[pl.ds(base, sup)], buf_v, sem).wait()
                    for j in range(grp):
                        pltpu.sync_copy(
                            buf_v.at[pl.ds(j * chunk, chunk)],
                            acc_s.at[rel_v.at[it * grp + j]], add=True)

                return c2

            lax.fori_loop(0, iters, step, 0)
            plsc.subcore_barrier()

            def wstep(it, c2):
                wc = s + it * NS

                @pl.when(wc < nwb)
                def _():
                    pltpu.sync_copy(acc_s.at[pl.ds(wc * _GR_WB, _GR_WB)],
                                    wb_v)
                    pltpu.sync_copy(
                        wb_v, out_h.at[pl.ds(g * N_GRID + rbase
                                             + wc * _GR_WB, _GR_WB)])

                return c2

            lax.fori_loop(0, witers, wstep, 0)
            plsc.subcore_barrier()

    return k(*vals8, dst)


def kernel(x, mesh_x, g2m_efeat, mesh_efeat, m2g_efeat,
           g2m_src, g2m_dst, mesh_src, mesh_dst, m2g_src, m2g_dst, params):
    p = params
    xf = x[:, 0, :, :].reshape(x.shape[0], -1).T

    grid = _embed(xf, p["grid_embed"], 480)
    mesh_xp = jnp.pad(mesh_x, ((0, M_PAD - N_MESH), (0, 0)))
    mesh_f = _embed(mesh_xp, p["mesh_embed"], 576)
    e_g2m = _embed(g2m_efeat, p["g2m_e_embed"], 640)
    e_mesh = _embed(mesh_efeat, p["mesh_e_embed"], 512)
    e_m2g = _embed(m2g_efeat, p["m2g_e_embed"], 480)

    ep = p["g2m_edge"]
    pg = _proj(grid, ep["w1"][HID:2 * HID], 480)
    pm = _proj(mesh_f, ep["w1"][2 * HID:], 576)
    gs = _sc_gather_sum(pg, pm, g2m_src, g2m_dst, 128)
    e_g2m = _edge_stage(e_g2m, gs, ep, 640)
    agg2 = _sc_segsum_mesh(e_g2m, g2m_dst, 128)
    mesh_f = _node_stage([agg2[:M_PAD], agg2[M_PAD:]], mesh_f,
                         p["g2m_node"], 576)
    grid = _res_mlp(grid, p["g2m_grid"], 480)

    for lp in p["proc"]:
        ew = lp["edge"]
        ps, pd = _proj2(mesh_f, ew["w1"][HID:2 * HID], ew["w1"][2 * HID:], 576)
        gs = _sc_gather_sum(ps, pd, mesh_src, mesh_dst, 128)
        e_mesh = _edge_stage(e_mesh, gs, ew, 512)
        agg2 = _sc_segsum_mesh(e_mesh, mesh_dst, 128)
        mesh_f = _node_stage([agg2[:M_PAD], agg2[M_PAD:]], mesh_f,
                             lp["node"], 576)

    ep = p["m2g_edge"]
    ps = _proj(mesh_f, ep["w1"][HID:2 * HID], 576)
    pd = _proj(grid, ep["w1"][2 * HID:], 480)
    gs = _sc_gather_sum(ps, pd, m2g_src, m2g_dst, 96)
    e8 = _edge_stage_split(e_m2g, gs, ep, 480, _NGRP)
    aggp = _sc_segsum_grid(e8, m2g_dst, 96, 9)
    parts = [aggp[g * N_GRID:(g + 1) * N_GRID] for g in range(_NGRP)]
    grid = _node_stage_cat(parts, grid, p["m2g_node"], 480)

    out = _decoder(grid, p["dec"], 480)[:, :1]
    return out.reshape(W, H, 1).transpose(2, 0, 1)[None]
